# baseline (speedup 1.0000x reference)
import functools

import jax
import jax.numpy as jnp
from jax import lax
from jax.experimental import pallas as pl
from jax.experimental.pallas import tpu as pltpu
from jax.experimental.pallas import tpu_sc as plsc

N = 10000
E = 160000
D = 256
G = 128
L = 5
T = 1


def _lin1_body(h, agg, eps, w1, b1, o, mo):
    z = (1.0 + eps[0, 0]) * h[...] + agg[...]
    z1 = jnp.dot(z, w1[...]) + b1[...]
    o[...] = z1
    mo[...] = jnp.mean(z1, axis=0, keepdims=True)


def _lin1(h, agg, eps, w1, b1):
    return pl.pallas_call(
        _lin1_body,
        out_shape=(jax.ShapeDtypeStruct((N, D), jnp.float32),
                   jax.ShapeDtypeStruct((1, D), jnp.float32)),
    )(h, agg, eps, w1, b1)


def _lin2_body(z1, m1, s1, g1, bt1, w2, b2, o, mo):
    zn = (z1[...] - m1[...]) / s1[...] * g1[...] + bt1[...]
    zn = jnp.maximum(zn, 0.0)
    z2 = jnp.dot(zn, w2[...]) + b2[...]
    o[...] = z2
    mo[...] = jnp.mean(z2, axis=0, keepdims=True)


def _lin2(z1, m1, s1, g1, bt1, w2, b2):
    return pl.pallas_call(
        _lin2_body,
        out_shape=(jax.ShapeDtypeStruct((N, D), jnp.float32),
                   jax.ShapeDtypeStruct((1, D), jnp.float32)),
    )(z1, m1, s1, g1, bt1, w2, b2)


def _norm_body(z2, m2, s2, g2, bt2, o, *, relu):
    zn = (z2[...] - m2[...]) / s2[...] * g2[...] + bt2[...]
    if relu:
        zn = jnp.maximum(zn, 0.0)
    o[...] = zn


def _norm(z2, m2, s2, g2, bt2, relu):
    return pl.pallas_call(
        functools.partial(_norm_body, relu=relu),
        out_shape=jax.ShapeDtypeStruct((N, D), jnp.float32),
    )(z2, m2, s2, g2, bt2)


def kernel(x, edge_index, edge_attr, batch, params):
    h = params['atom_table'][x]
    e_emb = params['bond_table'][edge_attr]
    src = edge_index[0]
    dst = edge_index[1]
    for l in range(L):
        lp = params['layers'][l]
        msg = jax.nn.relu(h[src] + e_emb)
        agg = jnp.zeros_like(h).at[dst].add(msg)
        eps = lp['eps'].reshape(1, 1).astype(jnp.float32)
        z1, m1 = _lin1(h, agg, eps, lp['W1'], lp['b1'].reshape(1, D))
        v1 = jnp.mean((z1 - m1) ** 2, axis=0, keepdims=True)
        s1 = jnp.sqrt(v1 + 1e-5)
        z2, m2 = _lin2(z1, m1, s1,
                       lp['g1'].reshape(1, D), lp['beta1'].reshape(1, D),
                       lp['W2'], lp['b2'].reshape(1, D))
        v2 = jnp.mean((z2 - m2) ** 2, axis=0, keepdims=True)
        s2 = jnp.sqrt(v2 + 1e-5)
        h = _norm(z2, m2, s2,
                  lp['gbn'].reshape(1, D), lp['bbn'].reshape(1, D),
                  relu=(l < L - 1))
    sums = jax.ops.segment_sum(h, batch, num_segments=G)
    cnt = jax.ops.segment_sum(jnp.ones((N, 1), dtype=h.dtype), batch, num_segments=G)
    out = (sums / jnp.maximum(cnt, 1.0)) @ params['pred_W'] + params['pred_b']
    return out.reshape(-1)
